# Initial kernel scaffold; baseline (speedup 1.0000x reference)
import jax
import jax.numpy as jnp
from jax.experimental import pallas as pl


def kernel(feature_indices, batch_idx, tables):
    raise NotImplementedError("write your pallas kernel here")


# speedup vs baseline: 1.9171x; 1.9171x over previous
import functools

import jax
import jax.numpy as jnp
from jax import lax
from jax.experimental import pallas as pl
from jax.experimental.pallas import tpu as pltpu
from jax.experimental.pallas import tpu_sc as plsc

B = 8
MAX_LEN = 2048
TOTAL = 8192
N_FEAT = 4
CARD_P2 = 1026
D = 256

NC = 2
NS = 16
L = 16
NW = NC * NS
TOK_PER_W = TOTAL // NW
CHUNK = 64
N_CHUNK = TOK_PER_W // CHUNK
GROW = N_FEAT * CHUNK
HALF = GROW // 2

_R = TOTAL // 128


def _prep_body(bidx_ref, dst_ref, pad_ref, mask_ref):
    b3 = bidx_ref[...]
    r = lax.broadcasted_iota(jnp.int32, (_R, 128), 0)
    c = lax.broadcasted_iota(jnp.int32, (_R, 128), 1)
    t = r * 128 + c

    counts = [jnp.sum((b3 == b).astype(jnp.int32)) for b in range(B)]
    starts = []
    s = jnp.int32(0)
    for b in range(B):
        starts.append(s)
        s = s + counts[b]

    ssel = jnp.zeros((_R, 128), jnp.int32)
    for b in range(B):
        ssel = jnp.where(b3 == b, starts[b], ssel)
    dst_ref[...] = b3 * MAX_LEN + (t - ssel)

    cpad = []
    cp = jnp.int32(0)
    for b in range(B):
        cpad.append(cp)
        cp = cp + (MAX_LEN - counts[b])
    bk = jnp.zeros((_R, 128), jnp.int32)
    for b in range(1, B):
        bk = bk + (t >= cpad[b]).astype(jnp.int32)
    csel = jnp.zeros((_R, 128), jnp.int32)
    cpsel = jnp.zeros((_R, 128), jnp.int32)
    for b in range(B):
        csel = jnp.where(bk == b, counts[b], csel)
        cpsel = jnp.where(bk == b, cpad[b], cpsel)
    pad_ref[...] = bk * MAX_LEN + csel + (t - cpsel)

    j = lax.broadcasted_iota(jnp.int32, (B, MAX_LEN), 1)
    row = lax.broadcasted_iota(jnp.int32, (B, MAX_LEN), 0)
    cm = jnp.zeros((B, MAX_LEN), jnp.int32)
    for b in range(B):
        cm = jnp.where(row == b, counts[b], cm)
    mask_ref[...] = j >= cm


_prep = pl.pallas_call(
    _prep_body,
    out_shape=(
        jax.ShapeDtypeStruct((_R, 128), jnp.int32),
        jax.ShapeDtypeStruct((_R, 128), jnp.int32),
        jax.ShapeDtypeStruct((B, MAX_LEN), jnp.bool_),
    ),
)

def _sc_body(fi_hbm, tab_hbm, dst_hbm, pad_hbm, out_hbm,
                      fiv, idxa, idxb, ga, gb, acc, zeros, dstv, padv,
                      sem0, sem1):
    wid = lax.axis_index("s") * NC + lax.axis_index("c")
    base = wid * TOK_PER_W

    def zfill(i, _):
        zeros[i // (D // L), pl.ds((i % (D // L)) * L, L)] = jnp.zeros(
            (L,), jnp.float32)
        return 0

    lax.fori_loop(0, CHUNK * (D // L), zfill, 0)

    offpat = (lax.iota(jnp.int32, L) % N_FEAT) * CARD_P2 + 1

    for ci in range(N_CHUNK):
        cbase = base + ci * CHUNK
        pltpu.sync_copy(fi_hbm.at[pl.ds(cbase * N_FEAT, GROW)], fiv)
        pltpu.sync_copy(dst_hbm.at[pl.ds(cbase, CHUNK)], dstv)
        pltpu.sync_copy(pad_hbm.at[pl.ds(cbase, CHUNK)], padv)
        for h in range(GROW // L):
            dst_buf = idxa if h < HALF // L else idxb
            o = (h % (HALF // L)) * L
            dst_buf[pl.ds(o, L)] = fiv[pl.ds(h * L, L)] + offpat
        cp_a = pltpu.async_copy(tab_hbm.at[idxa], ga, sem0)
        cp_b = pltpu.async_copy(tab_hbm.at[idxb], gb, sem1)
        cp_a.wait()
        cp_b.wait()

        def addbody_a(i, _):
            t = i // (D // L)
            co = (i % (D // L)) * L
            acc[t, pl.ds(co, L)] = (
                ga[4 * t, pl.ds(co, L)] + ga[4 * t + 1, pl.ds(co, L)]
                + ga[4 * t + 2, pl.ds(co, L)] + ga[4 * t + 3, pl.ds(co, L)])
            return 0

        def addbody_b(i, _):
            t = i // (D // L)
            co = (i % (D // L)) * L
            acc[t + CHUNK // 2, pl.ds(co, L)] = (
                gb[4 * t, pl.ds(co, L)] + gb[4 * t + 1, pl.ds(co, L)]
                + gb[4 * t + 2, pl.ds(co, L)] + gb[4 * t + 3, pl.ds(co, L)])
            return 0

        lax.fori_loop(0, (CHUNK // 2) * (D // L), addbody_a, 0)
        lax.fori_loop(0, (CHUNK // 2) * (D // L), addbody_b, 0)

        sc0 = pltpu.async_copy(acc, out_hbm.at[dstv], sem0)
        sc1 = pltpu.async_copy(zeros, out_hbm.at[padv], sem1)
        sc0.wait()
        sc1.wait()


@functools.lru_cache(maxsize=None)
def _build_sc():
    mesh = plsc.VectorSubcoreMesh(
        core_axis_name="c", subcore_axis_name="s",
        num_cores=NC, num_subcores=NS)
    return pl.kernel(
        _sc_body,
        out_type=jax.ShapeDtypeStruct((B * MAX_LEN, D), jnp.float32),
        mesh=mesh,
        scratch_types=[
            pltpu.VMEM((GROW,), jnp.int32),
            pltpu.VMEM((HALF,), jnp.int32),
            pltpu.VMEM((HALF,), jnp.int32),
            pltpu.VMEM((HALF, D), jnp.float32),
            pltpu.VMEM((HALF, D), jnp.float32),
            pltpu.VMEM((CHUNK, D), jnp.float32),
            pltpu.VMEM((CHUNK, D), jnp.float32),
            pltpu.VMEM((CHUNK,), jnp.int32),
            pltpu.VMEM((CHUNK,), jnp.int32),
            pltpu.SemaphoreType.DMA,
            pltpu.SemaphoreType.DMA,
        ],
    )


def kernel(feature_indices, batch_idx, tables):
    fi_flat = feature_indices.reshape(-1)
    tab_flat = tables.reshape(N_FEAT * CARD_P2, D)
    bidx3 = batch_idx.reshape(_R, 128)
    dst3, pad3, mask = _prep(bidx3)
    out2 = _build_sc()(fi_flat, tab_flat,
                       dst3.reshape(-1), pad3.reshape(-1))
    return out2.reshape(B, MAX_LEN, D), mask


# speedup vs baseline: 2.0565x; 1.0727x over previous
import functools

import jax
import jax.numpy as jnp
from jax import lax
from jax.experimental import pallas as pl
from jax.experimental.pallas import tpu as pltpu
from jax.experimental.pallas import tpu_sc as plsc

B = 8
MAX_LEN = 2048
TOTAL = 8192
N_FEAT = 4
CARD_P2 = 1026
D = 256

NC = 2
NS = 16
L = 16
NW = NC * NS
TOK_PER_W = TOTAL // NW
CHUNK = 64
N_CHUNK = TOK_PER_W // CHUNK
GROW = N_FEAT * CHUNK
HALF = GROW // 2

_R = TOTAL // 128


def _prep_body(bidx_ref, dst_ref, pad_ref, mask_ref):
    b3 = bidx_ref[...]
    r = lax.broadcasted_iota(jnp.int32, (_R, 128), 0)
    c = lax.broadcasted_iota(jnp.int32, (_R, 128), 1)
    t = r * 128 + c

    counts = [jnp.sum((b3 == b).astype(jnp.int32)) for b in range(B)]
    starts = []
    s = jnp.int32(0)
    for b in range(B):
        starts.append(s)
        s = s + counts[b]

    ssel = jnp.zeros((_R, 128), jnp.int32)
    for b in range(B):
        ssel = jnp.where(b3 == b, starts[b], ssel)
    dst_ref[...] = b3 * MAX_LEN + (t - ssel)

    cpad = []
    cp = jnp.int32(0)
    for b in range(B):
        cpad.append(cp)
        cp = cp + (MAX_LEN - counts[b])
    bk = jnp.zeros((_R, 128), jnp.int32)
    for b in range(1, B):
        bk = bk + (t >= cpad[b]).astype(jnp.int32)
    csel = jnp.zeros((_R, 128), jnp.int32)
    cpsel = jnp.zeros((_R, 128), jnp.int32)
    for b in range(B):
        csel = jnp.where(bk == b, counts[b], csel)
        cpsel = jnp.where(bk == b, cpad[b], cpsel)
    pad_ref[...] = bk * MAX_LEN + csel + (t - cpsel)

    j = lax.broadcasted_iota(jnp.int32, (B, MAX_LEN), 1)
    row = lax.broadcasted_iota(jnp.int32, (B, MAX_LEN), 0)
    cm = jnp.zeros((B, MAX_LEN), jnp.int32)
    for b in range(B):
        cm = jnp.where(row == b, counts[b], cm)
    mask_ref[...] = j >= cm


_prep = pl.pallas_call(
    _prep_body,
    out_shape=(
        jax.ShapeDtypeStruct((_R, 128), jnp.int32),
        jax.ShapeDtypeStruct((_R, 128), jnp.int32),
        jax.ShapeDtypeStruct((B, MAX_LEN), jnp.bool_),
    ),
)

def _sc_body(fi_hbm, tab_hbm, dst_hbm, pad_hbm, out_hbm,
             fiv, idxa, idxb, ga, gb, acc0, acc1, zeros,
             dv0, dv1, dv2, dv3, pv0, pv1, pv2, pv3,
             sem0, sem1, sem2, sem3, sem4):
    wid = lax.axis_index("s") * NC + lax.axis_index("c")
    base = wid * TOK_PER_W
    dvs = [dv0, dv1, dv2, dv3]
    pvs = [pv0, pv1, pv2, pv3]
    accs = [acc0, acc1]

    stage = [pltpu.async_copy(
        fi_hbm.at[pl.ds(base * N_FEAT, TOK_PER_W * N_FEAT)], fiv, sem2)]
    for ci in range(N_CHUNK):
        cb = base + ci * CHUNK
        stage.append(pltpu.async_copy(
            dst_hbm.at[pl.ds(cb, CHUNK)], dvs[ci], sem2))
        stage.append(pltpu.async_copy(
            pad_hbm.at[pl.ds(cb, CHUNK)], pvs[ci], sem2))

    zvec = jnp.zeros((L,), jnp.float32)

    def zfill(t, _):
        for c in range(D // L):
            zeros[t, pl.ds(c * L, L)] = zvec
        return 0

    lax.fori_loop(0, CHUNK, zfill, 0)
    for cp in stage:
        cp.wait()

    offpat = (lax.iota(jnp.int32, L) % N_FEAT) * CARD_P2 + 1

    prev_sc = None
    for ci in range(N_CHUNK):
        fo = ci * GROW
        for h in range(GROW // L):
            dst_buf = idxa if h < HALF // L else idxb
            o = (h % (HALF // L)) * L
            dst_buf[pl.ds(o, L)] = fiv[pl.ds(fo + h * L, L)] + offpat
        cp_a = pltpu.async_copy(tab_hbm.at[idxa], ga, sem0)
        cp_b = pltpu.async_copy(tab_hbm.at[idxb], gb, sem1)
        cp_a.wait()
        cp_b.wait()

        acc = accs[ci % 2]

        def addbody_a(t, _):
            for c in range(D // L):
                co = c * L
                acc[t, pl.ds(co, L)] = (
                    ga[4 * t, pl.ds(co, L)] + ga[4 * t + 1, pl.ds(co, L)]
                    + ga[4 * t + 2, pl.ds(co, L)]
                    + ga[4 * t + 3, pl.ds(co, L)])
            return 0

        def addbody_b(t, _):
            for c in range(D // L):
                co = c * L
                acc[t + CHUNK // 2, pl.ds(co, L)] = (
                    gb[4 * t, pl.ds(co, L)] + gb[4 * t + 1, pl.ds(co, L)]
                    + gb[4 * t + 2, pl.ds(co, L)]
                    + gb[4 * t + 3, pl.ds(co, L)])
            return 0

        lax.fori_loop(0, CHUNK // 2, addbody_a, 0)
        lax.fori_loop(0, CHUNK // 2, addbody_b, 0)

        if prev_sc is not None:
            for cp in prev_sc:
                cp.wait()
        prev_sc = (pltpu.async_copy(acc, out_hbm.at[dvs[ci]], sem3),
                   pltpu.async_copy(zeros, out_hbm.at[pvs[ci]], sem4))
    for cp in prev_sc:
        cp.wait()


@functools.lru_cache(maxsize=None)
def _build_sc():
    mesh = plsc.VectorSubcoreMesh(
        core_axis_name="c", subcore_axis_name="s",
        num_cores=NC, num_subcores=NS)
    return pl.kernel(
        _sc_body,
        out_type=jax.ShapeDtypeStruct((B * MAX_LEN, D), jnp.float32),
        mesh=mesh,
        scratch_types=[
            pltpu.VMEM((TOK_PER_W * N_FEAT,), jnp.int32),
            pltpu.VMEM((HALF,), jnp.int32),
            pltpu.VMEM((HALF,), jnp.int32),
            pltpu.VMEM((HALF, D), jnp.float32),
            pltpu.VMEM((HALF, D), jnp.float32),
            pltpu.VMEM((CHUNK, D), jnp.float32),
            pltpu.VMEM((CHUNK, D), jnp.float32),
            pltpu.VMEM((CHUNK, D), jnp.float32),
        ] + [pltpu.VMEM((CHUNK,), jnp.int32) for _ in range(2 * N_CHUNK)]
        + [pltpu.SemaphoreType.DMA for _ in range(5)],
    )


def kernel(feature_indices, batch_idx, tables):
    fi_flat = feature_indices.reshape(-1)
    tab_flat = tables.reshape(N_FEAT * CARD_P2, D)
    bidx3 = batch_idx.reshape(_R, 128)
    dst3, pad3, mask = _prep(bidx3)
    out2 = _build_sc()(fi_flat, tab_flat,
                       dst3.reshape(-1), pad3.reshape(-1))
    return out2.reshape(B, MAX_LEN, D), mask


# speedup vs baseline: 2.3966x; 1.1653x over previous
import functools

import jax
import jax.numpy as jnp
from jax import lax
from jax.experimental import pallas as pl
from jax.experimental.pallas import tpu as pltpu
from jax.experimental.pallas import tpu_sc as plsc

B = 8
MAX_LEN = 2048
TOTAL = 8192
N_FEAT = 4
CARD_P2 = 1026
D = 256

NC = 2
NS = 16
L = 16
NW = NC * NS
TOK_PER_W = TOTAL // NW
CHUNK = 32
N_CHUNK = TOK_PER_W // CHUNK
GROW = N_FEAT * CHUNK

_R = TOTAL // 128


def _prep_body(bidx_ref, dst_ref, pad_ref, mask_ref):
    b3 = bidx_ref[...]
    r = lax.broadcasted_iota(jnp.int32, (_R, 128), 0)
    c = lax.broadcasted_iota(jnp.int32, (_R, 128), 1)
    t = r * 128 + c

    counts = [jnp.sum((b3 == b).astype(jnp.int32)) for b in range(B)]
    starts = []
    s = jnp.int32(0)
    for b in range(B):
        starts.append(s)
        s = s + counts[b]

    ssel = jnp.zeros((_R, 128), jnp.int32)
    for b in range(B):
        ssel = jnp.where(b3 == b, starts[b], ssel)
    dst_ref[...] = b3 * MAX_LEN + (t - ssel)

    cpad = []
    cp = jnp.int32(0)
    for b in range(B):
        cpad.append(cp)
        cp = cp + (MAX_LEN - counts[b])
    bk = jnp.zeros((_R, 128), jnp.int32)
    for b in range(1, B):
        bk = bk + (t >= cpad[b]).astype(jnp.int32)
    csel = jnp.zeros((_R, 128), jnp.int32)
    cpsel = jnp.zeros((_R, 128), jnp.int32)
    for b in range(B):
        csel = jnp.where(bk == b, counts[b], csel)
        cpsel = jnp.where(bk == b, cpad[b], cpsel)
    pad_ref[...] = bk * MAX_LEN + csel + (t - cpsel)

    j = lax.broadcasted_iota(jnp.int32, (B, MAX_LEN), 1)
    row = lax.broadcasted_iota(jnp.int32, (B, MAX_LEN), 0)
    cm = jnp.zeros((B, MAX_LEN), jnp.int32)
    for b in range(B):
        cm = jnp.where(row == b, counts[b], cm)
    mask_ref[...] = j >= cm


_prep = pl.pallas_call(
    _prep_body,
    out_shape=(
        jax.ShapeDtypeStruct((_R, 128), jnp.int32),
        jax.ShapeDtypeStruct((_R, 128), jnp.int32),
        jax.ShapeDtypeStruct((B, MAX_LEN), jnp.bool_),
    ),
)


def _sc_body(fi_hbm, tab_hbm, dst_hbm, pad_hbm, out_hbm,
             fiv, idx0, idx1, g0, g1, acc0, acc1, zeros, dvm, pvm,
             semstage, semg0, semg1, semsc0, semsc1, semz):
    wid = lax.axis_index("s") * NC + lax.axis_index("c")

    stage = [
        pltpu.async_copy(fi_hbm.at[wid], fiv, semstage),
        pltpu.async_copy(dst_hbm.at[wid], dvm, semstage),
        pltpu.async_copy(pad_hbm.at[wid], pvm, semstage),
    ]

    zvec = jnp.zeros((L,), jnp.float32)

    def zfill(t, _):
        for c in range(D // L):
            zeros[t, pl.ds(c * L, L)] = zvec
        return 0

    lax.fori_loop(0, CHUNK, zfill, 0)
    for cpd in stage:
        cpd.wait()

    offpat = (lax.iota(jnp.int32, L) % N_FEAT) * CARD_P2 + 1

    idxs = [idx0, idx1]
    gs = [g0, g1]
    accs = [acc0, acc1]
    semgs = [semg0, semg1]
    semscs = [semsc0, semsc1]

    def build_idx(ci):
        buf = idxs[ci % 2]
        fo = ci * GROW
        for h in range(GROW // L):
            buf[pl.ds(h * L, L)] = fiv[pl.ds(fo + h * L, L)] + offpat

    def start_gather(ci):
        return pltpu.async_copy(
            tab_hbm.at[idxs[ci % 2]], gs[ci % 2], semgs[ci % 2])

    build_idx(0)
    gcp = {0: start_gather(0)}
    sc_tok = {}
    sc_zero = []
    for ci in range(N_CHUNK):
        par = ci % 2
        if ci + 1 < N_CHUNK:
            build_idx(ci + 1)
            gcp[ci + 1] = start_gather(ci + 1)
        gcp[ci].wait()
        if ci - 2 in sc_tok:
            sc_tok[ci - 2].wait()

        g = gs[par]
        acc = accs[par]

        def addbody(t, _):
            for c in range(D // L):
                co = c * L
                acc[t, pl.ds(co, L)] = (
                    g[4 * t, pl.ds(co, L)] + g[4 * t + 1, pl.ds(co, L)]
                    + g[4 * t + 2, pl.ds(co, L)] + g[4 * t + 3, pl.ds(co, L)])
            return 0

        lax.fori_loop(0, CHUNK, addbody, 0)

        sc_tok[ci] = pltpu.async_copy(acc, out_hbm.at[dvm.at[ci]], semscs[par])
        sc_zero.append(pltpu.async_copy(zeros, out_hbm.at[pvm.at[ci]], semz))
    for ci in (N_CHUNK - 2, N_CHUNK - 1):
        sc_tok[ci].wait()
    for cpd in sc_zero:
        cpd.wait()


@functools.lru_cache(maxsize=None)
def _build_sc():
    mesh = plsc.VectorSubcoreMesh(
        core_axis_name="c", subcore_axis_name="s",
        num_cores=NC, num_subcores=NS)
    return pl.kernel(
        _sc_body,
        out_type=jax.ShapeDtypeStruct((B * MAX_LEN, D), jnp.float32),
        mesh=mesh,
        scratch_types=[
            pltpu.VMEM((TOK_PER_W * N_FEAT,), jnp.int32),
            pltpu.VMEM((GROW,), jnp.int32),
            pltpu.VMEM((GROW,), jnp.int32),
            pltpu.VMEM((GROW, D), jnp.float32),
            pltpu.VMEM((GROW, D), jnp.float32),
            pltpu.VMEM((CHUNK, D), jnp.float32),
            pltpu.VMEM((CHUNK, D), jnp.float32),
            pltpu.VMEM((CHUNK, D), jnp.float32),
            pltpu.VMEM((N_CHUNK, CHUNK), jnp.int32),
            pltpu.VMEM((N_CHUNK, CHUNK), jnp.int32),
        ] + [pltpu.SemaphoreType.DMA for _ in range(6)],
    )


def kernel(feature_indices, batch_idx, tables):
    fi_w = feature_indices.reshape(NW, TOK_PER_W * N_FEAT)
    tab_flat = tables.reshape(N_FEAT * CARD_P2, D)
    bidx3 = batch_idx.reshape(_R, 128)
    dst3, pad3, mask = _prep(bidx3)
    dst_w = dst3.reshape(NW, N_CHUNK, CHUNK)
    pad_w = pad3.reshape(NW, N_CHUNK, CHUNK)
    out2 = _build_sc()(fi_w, tab_flat, dst_w, pad_w)
    return out2.reshape(B, MAX_LEN, D), mask


# speedup vs baseline: 2.4411x; 1.0186x over previous
import functools

import jax
import jax.numpy as jnp
from jax import lax
from jax.experimental import pallas as pl
from jax.experimental.pallas import tpu as pltpu
from jax.experimental.pallas import tpu_sc as plsc

B = 8
MAX_LEN = 2048
TOTAL = 8192
N_FEAT = 4
CARD_P2 = 1026
D = 256

NC = 2
NS = 16
L = 16
NW = NC * NS
TOK_PER_W = TOTAL // NW
CHUNK = 32
N_CHUNK = TOK_PER_W // CHUNK
GROW = N_FEAT * CHUNK

_GR = TOTAL * N_FEAT // 128


def _prep_body(bidx_ref, fi2_ref, gidx_ref, dst_ref, pad_ref, mask_ref):
    f2 = lax.broadcasted_iota(jnp.int32, (_GR, 128), 1) % N_FEAT
    gidx_ref[...] = fi2_ref[...] + 1 + f2 * CARD_P2

    b3 = bidx_ref[...]
    shp = (NW, N_CHUNK, CHUNK)
    i0 = lax.broadcasted_iota(jnp.int32, shp, 0)
    i1 = lax.broadcasted_iota(jnp.int32, shp, 1)
    i2 = lax.broadcasted_iota(jnp.int32, shp, 2)
    t = (i0 * N_CHUNK + i1) * CHUNK + i2

    counts = [jnp.sum((b3 == b).astype(jnp.int32)) for b in range(B)]
    starts = []
    s = jnp.int32(0)
    for b in range(B):
        starts.append(s)
        s = s + counts[b]

    ssel = jnp.zeros(shp, jnp.int32)
    for b in range(B):
        ssel = jnp.where(b3 == b, starts[b], ssel)
    dst_ref[...] = b3 * MAX_LEN + (t - ssel)

    cpad = []
    cp = jnp.int32(0)
    for b in range(B):
        cpad.append(cp)
        cp = cp + (MAX_LEN - counts[b])
    bk = jnp.zeros(shp, jnp.int32)
    for b in range(1, B):
        bk = bk + (t >= cpad[b]).astype(jnp.int32)
    csel = jnp.zeros(shp, jnp.int32)
    cpsel = jnp.zeros(shp, jnp.int32)
    for b in range(B):
        csel = jnp.where(bk == b, counts[b], csel)
        cpsel = jnp.where(bk == b, cpad[b], cpsel)
    pad_ref[...] = bk * MAX_LEN + csel + (t - cpsel)

    j = lax.broadcasted_iota(jnp.int32, (B, MAX_LEN), 1)
    row = lax.broadcasted_iota(jnp.int32, (B, MAX_LEN), 0)
    cm = jnp.zeros((B, MAX_LEN), jnp.int32)
    for b in range(B):
        cm = jnp.where(row == b, counts[b], cm)
    mask_ref[...] = j >= cm


_prep = pl.pallas_call(
    _prep_body,
    out_shape=(
        jax.ShapeDtypeStruct((_GR, 128), jnp.int32),
        jax.ShapeDtypeStruct((NW, N_CHUNK, CHUNK), jnp.int32),
        jax.ShapeDtypeStruct((NW, N_CHUNK, CHUNK), jnp.int32),
        jax.ShapeDtypeStruct((B, MAX_LEN), jnp.bool_),
    ),
)

_GPW = TOK_PER_W * N_FEAT // 128


def _sc_body(gidx_hbm, tab_hbm, dst_hbm, pad_hbm, out_hbm,
             gim, g0, g1, acc0, acc1, zeros, dvm, pvm,
             semstage, semg0, semg1, semsc0, semsc1, semz):
    wid = lax.axis_index("s") * NC + lax.axis_index("c")

    stage = [
        pltpu.async_copy(gidx_hbm.at[pl.ds(wid * _GPW, _GPW)], gim, semstage),
        pltpu.async_copy(dst_hbm.at[wid], dvm, semstage),
        pltpu.async_copy(pad_hbm.at[wid], pvm, semstage),
    ]

    zvec = jnp.zeros((L,), jnp.float32)

    def zfill(t, _):
        for c in range(D // L):
            zeros[t, pl.ds(c * L, L)] = zvec
        return 0

    lax.fori_loop(0, CHUNK, zfill, 0)
    for cpd in stage:
        cpd.wait()

    gs = [g0, g1]
    accs = [acc0, acc1]
    semgs = [semg0, semg1]
    semscs = [semsc0, semsc1]

    def start_gather(ci):
        return pltpu.async_copy(
            tab_hbm.at[gim.at[ci]], gs[ci % 2], semgs[ci % 2])

    gcp = {0: start_gather(0)}
    sc_tok = {}
    sc_zero = []
    for ci in range(N_CHUNK):
        par = ci % 2
        if ci + 1 < N_CHUNK:
            gcp[ci + 1] = start_gather(ci + 1)
        gcp[ci].wait()
        if ci - 2 in sc_tok:
            sc_tok[ci - 2].wait()

        g = gs[par]
        acc = accs[par]

        def addbody(t, _):
            for c in range(D // L):
                co = c * L
                acc[t, pl.ds(co, L)] = (
                    g[4 * t, pl.ds(co, L)] + g[4 * t + 1, pl.ds(co, L)]
                    + g[4 * t + 2, pl.ds(co, L)] + g[4 * t + 3, pl.ds(co, L)])
            return 0

        lax.fori_loop(0, CHUNK, addbody, 0)

        sc_tok[ci] = pltpu.async_copy(acc, out_hbm.at[dvm.at[ci]], semscs[par])
        sc_zero.append(pltpu.async_copy(zeros, out_hbm.at[pvm.at[ci]], semz))
    for ci in (N_CHUNK - 2, N_CHUNK - 1):
        sc_tok[ci].wait()
    for cpd in sc_zero:
        cpd.wait()


@functools.lru_cache(maxsize=None)
def _build_sc():
    mesh = plsc.VectorSubcoreMesh(
        core_axis_name="c", subcore_axis_name="s",
        num_cores=NC, num_subcores=NS)
    return pl.kernel(
        _sc_body,
        out_type=jax.ShapeDtypeStruct((B * MAX_LEN, D), jnp.float32),
        mesh=mesh,
        scratch_types=[
            pltpu.VMEM((_GPW, 128), jnp.int32),
            pltpu.VMEM((GROW, D), jnp.float32),
            pltpu.VMEM((GROW, D), jnp.float32),
            pltpu.VMEM((CHUNK, D), jnp.float32),
            pltpu.VMEM((CHUNK, D), jnp.float32),
            pltpu.VMEM((CHUNK, D), jnp.float32),
            pltpu.VMEM((N_CHUNK, CHUNK), jnp.int32),
            pltpu.VMEM((N_CHUNK, CHUNK), jnp.int32),
        ] + [pltpu.SemaphoreType.DMA for _ in range(6)],
    )


def kernel(feature_indices, batch_idx, tables):
    fi2 = feature_indices.reshape(_GR, 128)
    tab_flat = tables.reshape(N_FEAT * CARD_P2, D)
    bidx_w = batch_idx.reshape(NW, N_CHUNK, CHUNK)
    gidx, dst_w, pad_w, mask = _prep(bidx_w, fi2)
    out2 = _build_sc()(gidx, tab_flat, dst_w, pad_w)
    return out2.reshape(B, MAX_LEN, D), mask


# speedup vs baseline: 2.8654x; 1.1738x over previous
import functools

import jax
import jax.numpy as jnp
from jax import lax
from jax.experimental import pallas as pl
from jax.experimental.pallas import tpu as pltpu
from jax.experimental.pallas import tpu_sc as plsc

B = 8
MAX_LEN = 2048
TOTAL = 8192
N_FEAT = 4
CARD_P2 = 1026
D = 256

NC = 2
NS = 16
L = 16
NW = NC * NS
TOK_PER_W = TOTAL // NW
CHUNK = 32
N_CHUNK = TOK_PER_W // CHUNK
GROW = N_FEAT * CHUNK

_GR = TOTAL * N_FEAT // 128


def _prep_body(bidx_ref, fi2_ref, gidx_ref, dst_ref, pad_ref, mask_ref):
    f2 = lax.broadcasted_iota(jnp.int32, (_GR, 128), 1) % N_FEAT
    gidx_ref[...] = fi2_ref[...] + 1 + f2 * CARD_P2

    b3 = bidx_ref[...]
    shp = (NW, N_CHUNK, CHUNK)
    i0 = lax.broadcasted_iota(jnp.int32, shp, 0)
    i1 = lax.broadcasted_iota(jnp.int32, shp, 1)
    i2 = lax.broadcasted_iota(jnp.int32, shp, 2)
    t = (i0 * N_CHUNK + i1) * CHUNK + i2

    counts = [jnp.sum((b3 == b).astype(jnp.int32)) for b in range(B)]
    starts = []
    s = jnp.int32(0)
    for b in range(B):
        starts.append(s)
        s = s + counts[b]

    ssel = jnp.zeros(shp, jnp.int32)
    for b in range(B):
        ssel = jnp.where(b3 == b, starts[b], ssel)
    dst_ref[...] = b3 * MAX_LEN + (t - ssel)

    cpad = []
    cp = jnp.int32(0)
    for b in range(B):
        cpad.append(cp)
        cp = cp + (MAX_LEN - counts[b])
    bk = jnp.zeros(shp, jnp.int32)
    for b in range(1, B):
        bk = bk + (t >= cpad[b]).astype(jnp.int32)
    csel = jnp.zeros(shp, jnp.int32)
    cpsel = jnp.zeros(shp, jnp.int32)
    for b in range(B):
        csel = jnp.where(bk == b, counts[b], csel)
        cpsel = jnp.where(bk == b, cpad[b], cpsel)
    pad_ref[...] = bk * MAX_LEN + csel + (t - cpsel)

    j = lax.broadcasted_iota(jnp.int32, (B, MAX_LEN), 1)
    row = lax.broadcasted_iota(jnp.int32, (B, MAX_LEN), 0)
    cm = jnp.zeros((B, MAX_LEN), jnp.int32)
    for b in range(B):
        cm = jnp.where(row == b, counts[b], cm)
    mask_ref[...] = j >= cm


_prep = pl.pallas_call(
    _prep_body,
    out_shape=(
        jax.ShapeDtypeStruct((_GR, 128), jnp.int32),
        jax.ShapeDtypeStruct((NW, N_CHUNK, CHUNK), jnp.int32),
        jax.ShapeDtypeStruct((NW, N_CHUNK, CHUNK), jnp.int32),
        jax.ShapeDtypeStruct((B, MAX_LEN), jnp.bool_),
    ),
)

_GPW = TOK_PER_W * N_FEAT // 128


def _sc_body(gidx_hbm, tab_hbm, dst_hbm, pad_hbm, out_hbm,
             gim, g0, g1, acc0, acc1, zeros, dvm, pvm,
             semstage, semg0, semg1, semsc0, semsc1, semz):
    wid = lax.axis_index("s") * NC + lax.axis_index("c")

    stage = [
        pltpu.async_copy(gidx_hbm.at[pl.ds(wid * _GPW, _GPW)], gim, semstage),
        pltpu.async_copy(dst_hbm.at[wid], dvm, semstage),
        pltpu.async_copy(pad_hbm.at[wid], pvm, semstage),
    ]

    zvec = jnp.zeros((L,), jnp.float32)

    @plsc.parallel_loop(0, CHUNK, unroll=2)
    def _(t):
        for c in range(D // L):
            zeros[t, pl.ds(c * L, L)] = zvec
    for cpd in stage:
        cpd.wait()

    gs = [g0, g1]
    accs = [acc0, acc1]
    semgs = [semg0, semg1]
    semscs = [semsc0, semsc1]

    def start_gather(ci):
        return pltpu.async_copy(
            tab_hbm.at[gim.at[ci]], gs[ci % 2], semgs[ci % 2])

    gcp = {0: start_gather(0)}
    sc_tok = {}
    sc_zero = []
    for ci in range(N_CHUNK):
        par = ci % 2
        if ci + 1 < N_CHUNK:
            gcp[ci + 1] = start_gather(ci + 1)
        gcp[ci].wait()
        if ci - 2 in sc_tok:
            sc_tok[ci - 2].wait()

        g = gs[par]
        acc = accs[par]

        @plsc.parallel_loop(0, CHUNK, unroll=2)
        def _(t):
            for c in range(D // L):
                co = c * L
                acc[t, pl.ds(co, L)] = (
                    g[4 * t, pl.ds(co, L)] + g[4 * t + 1, pl.ds(co, L)]
                    + g[4 * t + 2, pl.ds(co, L)] + g[4 * t + 3, pl.ds(co, L)])

        sc_tok[ci] = pltpu.async_copy(acc, out_hbm.at[dvm.at[ci]], semscs[par])
        sc_zero.append(pltpu.async_copy(zeros, out_hbm.at[pvm.at[ci]], semz))
    for ci in (N_CHUNK - 2, N_CHUNK - 1):
        sc_tok[ci].wait()
    for cpd in sc_zero:
        cpd.wait()


@functools.lru_cache(maxsize=None)
def _build_sc():
    mesh = plsc.VectorSubcoreMesh(
        core_axis_name="c", subcore_axis_name="s",
        num_cores=NC, num_subcores=NS)
    return pl.kernel(
        _sc_body,
        out_type=jax.ShapeDtypeStruct((B * MAX_LEN, D), jnp.float32),
        mesh=mesh,
        scratch_types=[
            pltpu.VMEM((_GPW, 128), jnp.int32),
            pltpu.VMEM((GROW, D), jnp.float32),
            pltpu.VMEM((GROW, D), jnp.float32),
            pltpu.VMEM((CHUNK, D), jnp.float32),
            pltpu.VMEM((CHUNK, D), jnp.float32),
            pltpu.VMEM((CHUNK, D), jnp.float32),
            pltpu.VMEM((N_CHUNK, CHUNK), jnp.int32),
            pltpu.VMEM((N_CHUNK, CHUNK), jnp.int32),
        ] + [pltpu.SemaphoreType.DMA for _ in range(6)],
    )


def kernel(feature_indices, batch_idx, tables):
    fi2 = feature_indices.reshape(_GR, 128)
    tab_flat = tables.reshape(N_FEAT * CARD_P2, D)
    bidx_w = batch_idx.reshape(NW, N_CHUNK, CHUNK)
    gidx, dst_w, pad_w, mask = _prep(bidx_w, fi2)
    out2 = _build_sc()(gidx, tab_flat, dst_w, pad_w)
    return out2.reshape(B, MAX_LEN, D), mask


# speedup vs baseline: 2.9366x; 1.0249x over previous
import functools

import jax
import jax.numpy as jnp
from jax import lax
from jax.experimental import pallas as pl
from jax.experimental.pallas import tpu as pltpu
from jax.experimental.pallas import tpu_sc as plsc

B = 8
MAX_LEN = 2048
TOTAL = 8192
N_FEAT = 4
CARD_P2 = 1026
D = 256

NC = 2
NS = 16
L = 16
NW = NC * NS
TOK_PER_W = TOTAL // NW
CHUNK = 32
N_CHUNK = TOK_PER_W // CHUNK
GROW = N_FEAT * CHUNK

_GR = TOTAL * N_FEAT // 128


def _prep_body(bidx_ref, fi2_ref, gidx_ref, dst_ref, pad_ref, mask_ref):
    f2 = lax.broadcasted_iota(jnp.int32, (_GR, 128), 1) % N_FEAT
    gidx_ref[...] = fi2_ref[...] + 1 + f2 * CARD_P2

    b3 = bidx_ref[...]
    shp = (NW, N_CHUNK, CHUNK)
    i0 = lax.broadcasted_iota(jnp.int32, shp, 0)
    i1 = lax.broadcasted_iota(jnp.int32, shp, 1)
    i2 = lax.broadcasted_iota(jnp.int32, shp, 2)
    t = (i0 * N_CHUNK + i1) * CHUNK + i2

    counts = [jnp.sum((b3 == b).astype(jnp.int32)) for b in range(B)]
    starts = []
    s = jnp.int32(0)
    for b in range(B):
        starts.append(s)
        s = s + counts[b]

    ssel = jnp.zeros(shp, jnp.int32)
    for b in range(B):
        ssel = jnp.where(b3 == b, starts[b], ssel)
    dst_ref[...] = b3 * MAX_LEN + (t - ssel)

    cpad = []
    cp = jnp.int32(0)
    for b in range(B):
        cpad.append(cp)
        cp = cp + (MAX_LEN - counts[b])
    bk = jnp.zeros(shp, jnp.int32)
    for b in range(1, B):
        bk = bk + (t >= cpad[b]).astype(jnp.int32)
    csel = jnp.zeros(shp, jnp.int32)
    cpsel = jnp.zeros(shp, jnp.int32)
    for b in range(B):
        csel = jnp.where(bk == b, counts[b], csel)
        cpsel = jnp.where(bk == b, cpad[b], cpsel)
    pad_ref[...] = bk * MAX_LEN + csel + (t - cpsel)

    j = lax.broadcasted_iota(jnp.int32, (B, MAX_LEN), 1)
    row = lax.broadcasted_iota(jnp.int32, (B, MAX_LEN), 0)
    cm = jnp.zeros((B, MAX_LEN), jnp.int32)
    for b in range(B):
        cm = jnp.where(row == b, counts[b], cm)
    mask_ref[...] = j >= cm


_prep = pl.pallas_call(
    _prep_body,
    out_shape=(
        jax.ShapeDtypeStruct((_GR, 128), jnp.int32),
        jax.ShapeDtypeStruct((NW, N_CHUNK, CHUNK), jnp.int32),
        jax.ShapeDtypeStruct((NW, N_CHUNK, CHUNK), jnp.int32),
        jax.ShapeDtypeStruct((B, MAX_LEN), jnp.bool_),
    ),
)

_GPW = TOK_PER_W * N_FEAT // 128


def _sc_body(gidx_hbm, tab_hbm, dst_hbm, pad_hbm, out_hbm,
             gim, g0, g1, acc0, acc1, zeros, dvm, pvm,
             semstage, semg0, semg1, semsc0, semsc1, semz):
    wid = lax.axis_index("s") * NC + lax.axis_index("c")

    stage = [
        pltpu.async_copy(gidx_hbm.at[pl.ds(wid * _GPW, _GPW)], gim, semstage),
        pltpu.async_copy(dst_hbm.at[wid], dvm, semstage),
        pltpu.async_copy(pad_hbm.at[wid], pvm, semstage),
    ]

    zvec = jnp.zeros((L,), jnp.float32)

    @plsc.parallel_loop(0, CHUNK, unroll=4)
    def _(t):
        for c in range(D // L):
            zeros[t, pl.ds(c * L, L)] = zvec
    for cpd in stage:
        cpd.wait()

    gs = [g0, g1]
    accs = [acc0, acc1]
    semgs = [semg0, semg1]
    semscs = [semsc0, semsc1]

    def start_gather(ci):
        return pltpu.async_copy(
            tab_hbm.at[gim.at[ci]], gs[ci % 2], semgs[ci % 2])

    gcp = {0: start_gather(0)}
    sc_tok = {}
    sc_zero = []
    for ci in range(N_CHUNK):
        par = ci % 2
        if ci + 1 < N_CHUNK:
            gcp[ci + 1] = start_gather(ci + 1)
        gcp[ci].wait()
        if ci - 2 in sc_tok:
            sc_tok[ci - 2].wait()

        g = gs[par]
        acc = accs[par]

        @plsc.parallel_loop(0, CHUNK, unroll=4)
        def _(t):
            for c in range(D // L):
                co = c * L
                acc[t, pl.ds(co, L)] = (
                    g[4 * t, pl.ds(co, L)] + g[4 * t + 1, pl.ds(co, L)]
                    + g[4 * t + 2, pl.ds(co, L)] + g[4 * t + 3, pl.ds(co, L)])

        sc_tok[ci] = pltpu.async_copy(acc, out_hbm.at[dvm.at[ci]], semscs[par])
        sc_zero.append(pltpu.async_copy(zeros, out_hbm.at[pvm.at[ci]], semz))
    for ci in (N_CHUNK - 2, N_CHUNK - 1):
        sc_tok[ci].wait()
    for cpd in sc_zero:
        cpd.wait()


@functools.lru_cache(maxsize=None)
def _build_sc():
    mesh = plsc.VectorSubcoreMesh(
        core_axis_name="c", subcore_axis_name="s",
        num_cores=NC, num_subcores=NS)
    return pl.kernel(
        _sc_body,
        out_type=jax.ShapeDtypeStruct((B * MAX_LEN, D), jnp.float32),
        mesh=mesh,
        scratch_types=[
            pltpu.VMEM((_GPW, 128), jnp.int32),
            pltpu.VMEM((GROW, D), jnp.float32),
            pltpu.VMEM((GROW, D), jnp.float32),
            pltpu.VMEM((CHUNK, D), jnp.float32),
            pltpu.VMEM((CHUNK, D), jnp.float32),
            pltpu.VMEM((CHUNK, D), jnp.float32),
            pltpu.VMEM((N_CHUNK, CHUNK), jnp.int32),
            pltpu.VMEM((N_CHUNK, CHUNK), jnp.int32),
        ] + [pltpu.SemaphoreType.DMA for _ in range(6)],
    )


def kernel(feature_indices, batch_idx, tables):
    fi2 = feature_indices.reshape(_GR, 128)
    tab_flat = tables.reshape(N_FEAT * CARD_P2, D)
    bidx_w = batch_idx.reshape(NW, N_CHUNK, CHUNK)
    gidx, dst_w, pad_w, mask = _prep(bidx_w, fi2)
    out2 = _build_sc()(gidx, tab_flat, dst_w, pad_w)
    return out2.reshape(B, MAX_LEN, D), mask
